# Initial kernel scaffold; baseline (speedup 1.0000x reference)
import jax
import jax.numpy as jnp
from jax.experimental import pallas as pl


def kernel(x, b0_0_w, b0_0_gamma, b0_0_beta, b0_0_mean, b0_0_var, b1_0_w, b1_0_gamma, b1_0_beta, b1_0_mean, b1_0_var, b1_1_w, b1_1_gamma, b1_1_beta, b1_1_mean, b1_1_var, b1_2_w, b1_2_gamma, b1_2_beta, b1_2_mean, b1_2_var, b2_0_w, b2_0_gamma, b2_0_beta, b2_0_mean, b2_0_var, b2_1_w, b2_1_gamma, b2_1_beta, b2_1_mean, b2_1_var, b2_2_w, b2_2_gamma, b2_2_beta, b2_2_mean, b2_2_var, b2_3_w, b2_3_gamma, b2_3_beta, b2_3_mean, b2_3_var, b2_4_w, b2_4_gamma, b2_4_beta, b2_4_mean, b2_4_var, b3_0_w, b3_0_gamma, b3_0_beta, b3_0_mean, b3_0_var):
    raise NotImplementedError("write your pallas kernel here")


# speedup vs baseline: 1.9180x; 1.9180x over previous
import functools

import jax
import jax.numpy as jnp
from jax import lax
from jax.experimental import pallas as pl
from jax.experimental.pallas import tpu as pltpu

_EPS = 1e-3
_VMEM_LIMIT = 48 * 1024 * 1024


def _fold_bn(gamma, beta, mean, var):
    inv_std = 1.0 / jnp.sqrt(var.astype(jnp.float32) + _EPS)
    scale = gamma.astype(jnp.float32) * inv_std
    bias = beta.astype(jnp.float32) - mean.astype(jnp.float32) * scale
    return scale, bias


def _mat1x1(w, scale):
    return jnp.transpose(w[:, :, 0, 0]).astype(jnp.float32) * scale[None, :]


def _taps(w, scale, span):
    t = w[:, :, :, 0] if span == 'H' else w[:, :, 0, :]
    t = jnp.transpose(t, (2, 1, 0)).astype(jnp.float32)
    return t * scale[None, None, :]


def _inception_kernel(x_ref, w12_ref, b12_ref, w03_ref,
                      w11_ref, b11_ref, w21_ref, b21_ref,
                      w22_ref, b22_ref, w23_ref, b23_ref,
                      w1f_ref, w2f_ref, bcm_ref, o_ref,
                      *, hh, ww, c0, c1s, c1f, c2f, c3):
    hw = hh * ww
    f32 = jnp.float32
    bf16 = jnp.bfloat16
    dn_tt = (((0,), (0,)), ((), ()))

    xb = x_ref[...].astype(bf16)

    stem = lax.dot_general(xb, w12_ref[...], dn_tt,
                           preferred_element_type=f32)
    stem = jnp.maximum(stem + b12_ref[...], 0.0).astype(bf16)

    out03 = lax.dot_general(w03_ref[...], xb, dn_tt,
                            preferred_element_type=f32)

    x0 = jnp.maximum(out03[:c0, :] + bcm_ref[0:c0, :], 0.0)
    o_ref[0:c0, :] = x0.astype(bf16).astype(f32)

    y3 = out03[c0:, :]
    li = lax.broadcasted_iota(jnp.int32, (c3, hw), 1)
    wi = li % ww
    hi = li // ww
    z1 = jnp.zeros((c3, 1), f32)
    s1 = jnp.concatenate([z1, y3, z1], axis=1)
    rowsum = (jnp.where(wi > 0, s1[:, 0:hw], 0.0)
              + s1[:, 1:hw + 1]
              + jnp.where(wi < ww - 1, s1[:, 2:hw + 2], 0.0))
    zw = jnp.zeros((c3, ww), f32)
    s2 = jnp.concatenate([zw, rowsum, zw], axis=1)
    colsum = s2[:, 0:hw] + s2[:, ww:ww + hw] + s2[:, 2 * ww:2 * ww + hw]
    inv_w = jnp.where((wi == 0) | (wi == ww - 1), 0.5, 1.0 / 3.0)
    inv_h = jnp.where((hi == 0) | (hi == hh - 1), 0.5, 1.0 / 3.0)
    cbase = c0 + c1f + c2f
    x3 = jnp.maximum(colsum * (inv_w * inv_h) + bcm_ref[cbase:cbase + c3, :],
                     0.0)
    o_ref[cbase:cbase + c3, :] = x3

    def tap_mid(act, w_ref, b_ref, span):
        n_rows, cin = act.shape
        cout = w_ref.shape[2]
        stride = 1 if span == 'W' else ww
        z = jnp.zeros((3 * stride, cin), bf16)
        slab = jnp.concatenate([z, act, z], axis=0)
        wcol = lax.broadcasted_iota(jnp.int32, (n_rows, cout), 0) % ww
        acc = None
        for t in range(7):
            sl = slab[t * stride:t * stride + n_rows, :]
            p = lax.dot_general(sl, w_ref[t], (((1,), (0,)), ((), ())),
                                preferred_element_type=f32)
            if span == 'W' and t != 3:
                d = t - 3
                p = jnp.where((wcol + d >= 0) & (wcol + d < ww), p, 0.0)
            acc = p if acc is None else acc + p
        return jnp.maximum(acc + b_ref[...], 0.0).astype(bf16)

    def tap_fin(act, w_ref, span):
        n_rows, cin = act.shape
        cout = w_ref.shape[2]
        stride = 1 if span == 'W' else ww
        z = jnp.zeros((3 * stride, cin), bf16)
        slab = jnp.concatenate([z, act, z], axis=0)
        wlan = lax.broadcasted_iota(jnp.int32, (cout, n_rows), 1) % ww
        acc = None
        for t in range(7):
            sl = slab[t * stride:t * stride + n_rows, :]
            p = lax.dot_general(w_ref[t], sl, (((0,), (1,)), ((), ())),
                                preferred_element_type=f32)
            if span == 'W' and t != 3:
                d = t - 3
                p = jnp.where((wlan + d >= 0) & (wlan + d < ww), p, 0.0)
            acc = p if acc is None else acc + p
        return acc

    a = tap_mid(stem[:, 0:c1s], w11_ref, b11_ref, 'W')
    x1 = tap_fin(a, w1f_ref, 'H')
    x1 = jnp.maximum(x1 + bcm_ref[c0:c0 + c1f, :], 0.0)
    o_ref[c0:c0 + c1f, :] = x1

    b = tap_mid(stem[:, c1s:], w21_ref, b21_ref, 'H')
    b = tap_mid(b, w22_ref, b22_ref, 'W')
    b = tap_mid(b, w23_ref, b23_ref, 'H')
    x2 = tap_fin(b, w2f_ref, 'W')
    x2 = jnp.maximum(x2 + bcm_ref[c0 + c1f:c0 + c1f + c2f, :], 0.0)
    o_ref[c0 + c1f:c0 + c1f + c2f, :] = x2


def kernel(x,
           b0_0_w, b0_0_gamma, b0_0_beta, b0_0_mean, b0_0_var,
           b1_0_w, b1_0_gamma, b1_0_beta, b1_0_mean, b1_0_var,
           b1_1_w, b1_1_gamma, b1_1_beta, b1_1_mean, b1_1_var,
           b1_2_w, b1_2_gamma, b1_2_beta, b1_2_mean, b1_2_var,
           b2_0_w, b2_0_gamma, b2_0_beta, b2_0_mean, b2_0_var,
           b2_1_w, b2_1_gamma, b2_1_beta, b2_1_mean, b2_1_var,
           b2_2_w, b2_2_gamma, b2_2_beta, b2_2_mean, b2_2_var,
           b2_3_w, b2_3_gamma, b2_3_beta, b2_3_mean, b2_3_var,
           b2_4_w, b2_4_gamma, b2_4_beta, b2_4_mean, b2_4_var,
           b3_0_w, b3_0_gamma, b3_0_beta, b3_0_mean, b3_0_var):
    n, cin, hh, ww = x.shape
    hw = hh * ww
    bf16 = jnp.bfloat16
    f32 = jnp.float32

    s00, a00 = _fold_bn(b0_0_gamma, b0_0_beta, b0_0_mean, b0_0_var)
    s10, a10 = _fold_bn(b1_0_gamma, b1_0_beta, b1_0_mean, b1_0_var)
    s11, a11 = _fold_bn(b1_1_gamma, b1_1_beta, b1_1_mean, b1_1_var)
    s12, a12 = _fold_bn(b1_2_gamma, b1_2_beta, b1_2_mean, b1_2_var)
    s20, a20 = _fold_bn(b2_0_gamma, b2_0_beta, b2_0_mean, b2_0_var)
    s21, a21 = _fold_bn(b2_1_gamma, b2_1_beta, b2_1_mean, b2_1_var)
    s22, a22 = _fold_bn(b2_2_gamma, b2_2_beta, b2_2_mean, b2_2_var)
    s23, a23 = _fold_bn(b2_3_gamma, b2_3_beta, b2_3_mean, b2_3_var)
    s24, a24 = _fold_bn(b2_4_gamma, b2_4_beta, b2_4_mean, b2_4_var)
    s30, a30 = _fold_bn(b3_0_gamma, b3_0_beta, b3_0_mean, b3_0_var)

    c0 = b0_0_w.shape[0]
    c1s = b1_0_w.shape[0]
    c2s = b2_0_w.shape[0]
    c1f = b1_2_w.shape[0]
    c2f = b2_4_w.shape[0]
    c3 = b3_0_w.shape[0]
    ctot = c0 + c1f + c2f + c3

    w12 = jnp.concatenate([_mat1x1(b1_0_w, s10), _mat1x1(b2_0_w, s20)],
                          axis=1).astype(bf16)
    b12 = jnp.concatenate([a10, a20]).reshape(1, c1s + c2s).astype(f32)
    w03 = jnp.concatenate([_mat1x1(b0_0_w, s00), _mat1x1(b3_0_w, s30)],
                          axis=1).astype(bf16)

    w11 = _taps(b1_1_w, s11, 'W').astype(bf16)
    b11 = a11.reshape(1, -1).astype(f32)
    w21 = _taps(b2_1_w, s21, 'H').astype(bf16)
    b21 = a21.reshape(1, -1).astype(f32)
    w22 = _taps(b2_2_w, s22, 'W').astype(bf16)
    b22 = a22.reshape(1, -1).astype(f32)
    w23 = _taps(b2_3_w, s23, 'H').astype(bf16)
    b23 = a23.reshape(1, -1).astype(f32)
    w1f = _taps(b1_2_w, s12, 'H').astype(bf16)
    w2f = _taps(b2_4_w, s24, 'W').astype(bf16)

    bcm = jnp.broadcast_to(
        jnp.concatenate([a00, a12, a24, a30])[:, None],
        (ctot, hw)).astype(f32)

    kfn = functools.partial(_inception_kernel, hh=hh, ww=ww, c0=c0,
                            c1s=c1s, c1f=c1f, c2f=c2f, c3=c3)
    const = lambda n_: (0, 0)
    const3 = lambda n_: (0, 0, 0)
    out = pl.pallas_call(
        kfn,
        out_shape=jax.ShapeDtypeStruct((n, ctot, hw), f32),
        grid_spec=pltpu.PrefetchScalarGridSpec(
            num_scalar_prefetch=0,
            grid=(n,),
            in_specs=[
                pl.BlockSpec((None, cin, hw), lambda n_: (n_, 0, 0)),
                pl.BlockSpec(w12.shape, const),
                pl.BlockSpec(b12.shape, const),
                pl.BlockSpec(w03.shape, const),
                pl.BlockSpec(w11.shape, const3),
                pl.BlockSpec(b11.shape, const),
                pl.BlockSpec(w21.shape, const3),
                pl.BlockSpec(b21.shape, const),
                pl.BlockSpec(w22.shape, const3),
                pl.BlockSpec(b22.shape, const),
                pl.BlockSpec(w23.shape, const3),
                pl.BlockSpec(b23.shape, const),
                pl.BlockSpec(w1f.shape, const3),
                pl.BlockSpec(w2f.shape, const3),
                pl.BlockSpec(bcm.shape, const),
            ],
            out_specs=pl.BlockSpec((None, ctot, hw), lambda n_: (n_, 0, 0))),
        compiler_params=pltpu.CompilerParams(
            dimension_semantics=("parallel",),
            vmem_limit_bytes=_VMEM_LIMIT),
    )(x.reshape(n, cin, hw), w12, b12, w03, w11, b11, w21, b21,
      w22, b22, w23, b23, w1f, w2f, bcm)
    return out.reshape(n, ctot, hh, ww)


# speedup vs baseline: 3.5530x; 1.8525x over previous
import functools

import jax
import jax.numpy as jnp
from jax import lax
from jax.experimental import pallas as pl
from jax.experimental.pallas import tpu as pltpu

_EPS = 1e-3
_VMEM_LIMIT = 56 * 1024 * 1024


def _fold_bn(gamma, beta, mean, var):
    inv_std = 1.0 / jnp.sqrt(var.astype(jnp.float32) + _EPS)
    scale = gamma.astype(jnp.float32) * inv_std
    bias = beta.astype(jnp.float32) - mean.astype(jnp.float32) * scale
    return scale, bias


def _mat1x1(w, scale):
    return jnp.transpose(w[:, :, 0, 0]).astype(jnp.float32) * scale[None, :]


def _taps(w, scale, span):
    t = w[:, :, :, 0] if span == 'H' else w[:, :, 0, :]
    t = jnp.transpose(t, (2, 1, 0)).astype(jnp.float32)
    return t * scale[None, None, :]


def _inception_kernel(x_ref, wall_ref, b12_ref, b0_ref, b3_ref,
                      w11_ref, b11_ref, w21_ref, b21_ref,
                      w22_ref, b22_ref, w23_ref, b23_ref,
                      w1f_ref, b1f_ref, w2f_ref, b2f_ref, o_ref,
                      *, hh, ww, bn, c0, c1s, c2s, c1f, c2f, c3):
    hw = hh * ww
    rows = hw * bn
    f32 = jnp.float32
    bf16 = jnp.bfloat16

    xb = x_ref[...].reshape(rows, x_ref.shape[-1])

    acc = lax.dot_general(xb, wall_ref[...], (((1,), (0,)), ((), ())),
                          preferred_element_type=f32)
    stem12 = jnp.maximum(acc[:, :c1s + c2s] + b12_ref[...], 0.0).astype(bf16)
    x0 = jnp.maximum(acc[:, c1s + c2s:c1s + c2s + c0] + b0_ref[...], 0.0)
    x0 = x0.astype(bf16).astype(f32)
    o_ref[:, :, 0:c0] = x0.reshape(hw, bn, c0)
    y3 = acc[:, c1s + c2s + c0:]

    pi = lax.broadcasted_iota(jnp.int32, (rows, c3), 0) // bn
    wi = pi % ww
    hi = pi // ww
    zw = jnp.zeros((bn, c3), f32)
    s1 = jnp.concatenate([zw, y3, zw], axis=0)
    rowsum = (jnp.where(wi > 0, s1[0:rows, :], 0.0)
              + s1[bn:bn + rows, :]
              + jnp.where(wi < ww - 1, s1[2 * bn:2 * bn + rows, :], 0.0))
    zh = jnp.zeros((ww * bn, c3), f32)
    s2 = jnp.concatenate([zh, rowsum, zh], axis=0)
    colsum = (s2[0:rows, :] + s2[ww * bn:ww * bn + rows, :]
              + s2[2 * ww * bn:2 * ww * bn + rows, :])
    inv_w = jnp.where((wi == 0) | (wi == ww - 1), 0.5, 1.0 / 3.0)
    inv_h = jnp.where((hi == 0) | (hi == hh - 1), 0.5, 1.0 / 3.0)
    x3 = jnp.maximum(colsum * (inv_w * inv_h) + b3_ref[...], 0.0)
    o_ref[:, :, c0 + c1f + c2f:] = x3.reshape(hw, bn, c3)

    def tap(act, w_ref, b_ref, span, out_f32):
        cin = act.shape[1]
        cout = w_ref.shape[2]
        stride = bn if span == 'W' else ww * bn
        z = jnp.zeros((3 * stride, cin), bf16)
        slab = jnp.concatenate([z, act, z], axis=0)
        wcol = (lax.broadcasted_iota(jnp.int32, (rows, cout), 0) // bn) % ww
        acc_t = None
        for t in range(7):
            sl = slab[t * stride:t * stride + rows, :]
            p = lax.dot_general(sl, w_ref[t], (((1,), (0,)), ((), ())),
                                preferred_element_type=f32)
            if span == 'W' and t != 3:
                d = t - 3
                p = jnp.where((wcol + d >= 0) & (wcol + d < ww), p, 0.0)
            acc_t = p if acc_t is None else acc_t + p
        r = jnp.maximum(acc_t + b_ref[...], 0.0)
        return r if out_f32 else r.astype(bf16)

    a = tap(stem12[:, 0:c1s], w11_ref, b11_ref, 'W', False)
    x1 = tap(a, w1f_ref, b1f_ref, 'H', True)
    o_ref[:, :, c0:c0 + c1f] = x1.reshape(hw, bn, c1f)

    b = tap(stem12[:, c1s:], w21_ref, b21_ref, 'H', False)
    b = tap(b, w22_ref, b22_ref, 'W', False)
    b = tap(b, w23_ref, b23_ref, 'H', False)
    x2 = tap(b, w2f_ref, b2f_ref, 'W', True)
    o_ref[:, :, c0 + c1f:c0 + c1f + c2f] = x2.reshape(hw, bn, c2f)


def kernel(x,
           b0_0_w, b0_0_gamma, b0_0_beta, b0_0_mean, b0_0_var,
           b1_0_w, b1_0_gamma, b1_0_beta, b1_0_mean, b1_0_var,
           b1_1_w, b1_1_gamma, b1_1_beta, b1_1_mean, b1_1_var,
           b1_2_w, b1_2_gamma, b1_2_beta, b1_2_mean, b1_2_var,
           b2_0_w, b2_0_gamma, b2_0_beta, b2_0_mean, b2_0_var,
           b2_1_w, b2_1_gamma, b2_1_beta, b2_1_mean, b2_1_var,
           b2_2_w, b2_2_gamma, b2_2_beta, b2_2_mean, b2_2_var,
           b2_3_w, b2_3_gamma, b2_3_beta, b2_3_mean, b2_3_var,
           b2_4_w, b2_4_gamma, b2_4_beta, b2_4_mean, b2_4_var,
           b3_0_w, b3_0_gamma, b3_0_beta, b3_0_mean, b3_0_var):
    n, cin, hh, ww = x.shape
    hw = hh * ww
    bn = 8 if n % 8 == 0 else n
    bf16 = jnp.bfloat16
    f32 = jnp.float32

    s00, a00 = _fold_bn(b0_0_gamma, b0_0_beta, b0_0_mean, b0_0_var)
    s10, a10 = _fold_bn(b1_0_gamma, b1_0_beta, b1_0_mean, b1_0_var)
    s11, a11 = _fold_bn(b1_1_gamma, b1_1_beta, b1_1_mean, b1_1_var)
    s12, a12 = _fold_bn(b1_2_gamma, b1_2_beta, b1_2_mean, b1_2_var)
    s20, a20 = _fold_bn(b2_0_gamma, b2_0_beta, b2_0_mean, b2_0_var)
    s21, a21 = _fold_bn(b2_1_gamma, b2_1_beta, b2_1_mean, b2_1_var)
    s22, a22 = _fold_bn(b2_2_gamma, b2_2_beta, b2_2_mean, b2_2_var)
    s23, a23 = _fold_bn(b2_3_gamma, b2_3_beta, b2_3_mean, b2_3_var)
    s24, a24 = _fold_bn(b2_4_gamma, b2_4_beta, b2_4_mean, b2_4_var)
    s30, a30 = _fold_bn(b3_0_gamma, b3_0_beta, b3_0_mean, b3_0_var)

    c0 = b0_0_w.shape[0]
    c1s = b1_0_w.shape[0]
    c2s = b2_0_w.shape[0]
    c1f = b1_2_w.shape[0]
    c2f = b2_4_w.shape[0]
    c3 = b3_0_w.shape[0]
    ctot = c0 + c1f + c2f + c3

    wall = jnp.concatenate(
        [_mat1x1(b1_0_w, s10), _mat1x1(b2_0_w, s20),
         _mat1x1(b0_0_w, s00), _mat1x1(b3_0_w, s30)], axis=1).astype(bf16)
    b12 = jnp.concatenate([a10, a20]).reshape(1, c1s + c2s).astype(f32)
    b0b = a00.reshape(1, c0).astype(f32)
    b3b = a30.reshape(1, c3).astype(f32)

    w11 = _taps(b1_1_w, s11, 'W').astype(bf16)
    b11 = a11.reshape(1, -1).astype(f32)
    w21 = _taps(b2_1_w, s21, 'H').astype(bf16)
    b21 = a21.reshape(1, -1).astype(f32)
    w22 = _taps(b2_2_w, s22, 'W').astype(bf16)
    b22 = a22.reshape(1, -1).astype(f32)
    w23 = _taps(b2_3_w, s23, 'H').astype(bf16)
    b23 = a23.reshape(1, -1).astype(f32)
    w1f = _taps(b1_2_w, s12, 'H').astype(bf16)
    b1f = a12.reshape(1, -1).astype(f32)
    w2f = _taps(b2_4_w, s24, 'W').astype(bf16)
    b2f = a24.reshape(1, -1).astype(f32)

    x_p = jnp.transpose(x, (2, 3, 0, 1)).reshape(hw, n, cin).astype(bf16)

    kfn = functools.partial(_inception_kernel, hh=hh, ww=ww, bn=bn, c0=c0,
                            c1s=c1s, c2s=c2s, c1f=c1f, c2f=c2f, c3=c3)
    const = lambda i: (0, 0)
    const3 = lambda i: (0, 0, 0)
    out = pl.pallas_call(
        kfn,
        out_shape=jax.ShapeDtypeStruct((hw, n, ctot), f32),
        grid_spec=pltpu.PrefetchScalarGridSpec(
            num_scalar_prefetch=0,
            grid=(n // bn,),
            in_specs=[
                pl.BlockSpec((hw, bn, cin), lambda i: (0, i, 0)),
                pl.BlockSpec(wall.shape, const),
                pl.BlockSpec(b12.shape, const),
                pl.BlockSpec(b0b.shape, const),
                pl.BlockSpec(b3b.shape, const),
                pl.BlockSpec(w11.shape, const3),
                pl.BlockSpec(b11.shape, const),
                pl.BlockSpec(w21.shape, const3),
                pl.BlockSpec(b21.shape, const),
                pl.BlockSpec(w22.shape, const3),
                pl.BlockSpec(b22.shape, const),
                pl.BlockSpec(w23.shape, const3),
                pl.BlockSpec(b23.shape, const),
                pl.BlockSpec(w1f.shape, const3),
                pl.BlockSpec(b1f.shape, const),
                pl.BlockSpec(w2f.shape, const3),
                pl.BlockSpec(b2f.shape, const),
            ],
            out_specs=pl.BlockSpec((hw, bn, ctot), lambda i: (0, i, 0))),
        compiler_params=pltpu.CompilerParams(
            dimension_semantics=("parallel",),
            vmem_limit_bytes=_VMEM_LIMIT),
    )(x_p, wall, b12, b0b, b3b, w11, b11, w21, b21,
      w22, b22, w23, b23, w1f, b1f, w2f, b2f)
    return jnp.transpose(out.reshape(hh, ww, n, ctot), (2, 3, 0, 1))
